# baseline (speedup 1.0000x reference)
import functools

import jax
import jax.numpy as jnp
from jax import lax
from jax.experimental import pallas as pl
from jax.experimental.pallas import tpu as pltpu
from jax.experimental.pallas import tpu_sc as plsc

_NC = 2
_NS = 16
_NW = _NC * _NS
_L = 16

_B = 4096
_T = 200
_N = 64
_R = _B * _T
_RW = _R // _NW
_IDXW = 100
_CB = 2
_CR = _CB * _T
_IDX_ROWS = _CR // _IDXW
_CHUNKS = _RW // _CR
_GRP = _N // _L


def _emb_body(tok_hbm, idx_hbm, pos_hbm, out_hbm, idx_v, rows_v, pos_v, sem):
    wid = lax.axis_index("s") * _NC + lax.axis_index("c")
    pltpu.sync_copy(pos_hbm.at[pl.ds(0, _T)], pos_v)

    def chunk_body(c, carry):
        irow = wid * (_RW // _IDXW) + c * _IDX_ROWS
        pltpu.sync_copy(idx_hbm.at[pl.ds(irow, _IDX_ROWS)], idx_v)
        copies = []
        for g in range(_IDX_ROWS):
            cp = pltpu.make_async_copy(
                tok_hbm.at[idx_v.at[g]],
                rows_v.at[pl.ds(g * _IDXW, _IDXW)],
                sem,
            )
            cp.start()
            copies.append(cp)
        for cp in copies:
            cp.wait()

        def add_body(r, acc):
            for g in range(_GRP):
                sl = pl.ds(g * _L, _L)
                p = pos_v[r, sl]
                for rep in range(_CB):
                    rr = r + rep * _T
                    rows_v[rr, sl] = rows_v[rr, sl] + p
            return acc

        lax.fori_loop(0, _T, add_body, 0)
        rowbase = wid * _RW + c * _CR
        pltpu.sync_copy(rows_v, out_hbm.at[pl.ds(rowbase, _CR)])
        return carry

    lax.fori_loop(0, _CHUNKS, chunk_body, 0)


@functools.partial(jax.jit, static_argnums=())
def kernel(idx, tok_emb, pos_emb):
    b, t = idx.shape
    n = tok_emb.shape[1]
    idx2d = idx.astype(jnp.int32).reshape(-1, _IDXW)
    mesh = plsc.VectorSubcoreMesh(core_axis_name="c", subcore_axis_name="s")
    emb = pl.kernel(
        _emb_body,
        out_type=jax.ShapeDtypeStruct((_R, _N), jnp.float32),
        mesh=mesh,
        scratch_types=[
            pltpu.VMEM((_IDX_ROWS, _IDXW), jnp.int32),
            pltpu.VMEM((_CR, _N), jnp.float32),
            pltpu.VMEM((_T, _N), jnp.float32),
            pltpu.SemaphoreType.DMA,
        ],
        compiler_params=pltpu.CompilerParams(use_tc_tiling_on_sc=False),
    )
    out = emb(tok_emb, idx2d, pos_emb)
    return out.reshape(b, t, n)
